# baseline (speedup 1.0000x reference)
import functools

import jax
import jax.numpy as jnp
from jax.experimental import pallas as pl
from jax.experimental.pallas import tpu as pltpu

CHEB_K = 3
TOP_K = 2


def _r32(v):
    return v.astype(jnp.bfloat16).astype(jnp.float32)


def _bdot(a, b):
    return jnp.dot(a.astype(jnp.bfloat16), b.astype(jnp.bfloat16),
                   preferred_element_type=jnp.float32)


def _embed_body(x_ref, emb_w_ref, emb_b_ref, pos_ref, out_ref, xt_ref, *, T):
    t = pl.program_id(1)
    xb = _r32(x_ref[0, 0])
    C = xb.shape[1]
    wr = _r32(emb_w_ref[...])
    val = xb[:, 0:1] * wr[0:1, :]
    for c in range(1, C):
        val = val + xb[:, c : c + 1] * wr[c : c + 1, :]
    val = val + emb_b_ref[0:1, :] + pos_ref[0]
    out_ref[0, 0] = val

    @pl.when(t == 0)
    def _init():
        xt_ref[0] = val

    @pl.when((t > 0) & (t < T - 1))
    def _acc():
        xt_ref[0] += val

    @pl.when(t == T - 1)
    def _fin():
        xt_ref[0] = (xt_ref[0] + val) / float(T)


def _expert_body(xt_ref, sup_ref, gate_w_ref, exp_w_ref, exp_b_ref,
                 o_ref, h_ref, t2_ref, *, n_sup, n_exp):
    f32 = jnp.float32
    b = pl.program_id(0)
    xt = xt_ref[0]
    N = xt.shape[0]

    @pl.when(b == 0)
    def _build_t2():
        row = jax.lax.broadcasted_iota(jnp.int32, (N, N), 0)
        col = jax.lax.broadcasted_iota(jnp.int32, (N, N), 1)
        eye = (row == col).astype(f32)
        for s in range(n_sup):
            A = sup_ref[s]
            t2_ref[s] = 2.0 * _bdot(A, A) - eye

    chunks = []
    for s in range(n_sup):
        A = sup_ref[s]
        z1 = _bdot(A, xt)
        z2 = _bdot(t2_ref[s], xt)
        chunks.extend([_r32(xt), z1, z2])
    xg = jnp.concatenate(chunks, axis=1)

    gate = _bdot(xt, gate_w_ref[...])
    iota = jax.lax.broadcasted_iota(jnp.int32, (N, n_exp), 1)
    m1 = jnp.max(gate, axis=1, keepdims=True)
    idx1 = jnp.min(jnp.where(gate == m1, iota, n_exp), axis=1, keepdims=True)
    masked = jnp.where(iota == idx1, -jnp.inf, gate)
    m2 = jnp.max(masked, axis=1, keepdims=True)
    idx2 = jnp.min(jnp.where(masked == m2, iota, n_exp), axis=1, keepdims=True)
    e1 = jnp.exp(m2 - m1)
    denom = 1.0 + e1
    w1 = 1.0 / denom
    w2 = e1 / denom

    o = jnp.zeros_like(xt)
    for e in range(n_exp):
        oe = _bdot(xg, exp_w_ref[e])
        oe = oe + exp_b_ref[e : e + 1, :]
        coef = jnp.where(idx1 == e, w1, 0.0) + jnp.where(idx2 == e, w2, 0.0)
        o = o + coef * oe
    o_ref[0] = o
    h_ref[0] = jnp.tanh(o)


@jax.jit
def kernel(x, y_cov, supports, emb_W, emb_b, pos_emb, gate_W, exp_W, exp_b):
    B, T, N, C = x.shape
    D = emb_W.shape[1]
    n_sup = supports.shape[0]
    n_exp = exp_W.shape[0]
    emb_b2 = emb_b.reshape(1, D)

    current_inputs, xt = pl.pallas_call(
        functools.partial(_embed_body, T=T),
        grid=(B, T),
        in_specs=[
            pl.BlockSpec((1, 1, N, C), lambda b, t: (b, t, 0, 0)),
            pl.BlockSpec((C, D), lambda b, t: (0, 0)),
            pl.BlockSpec((1, D), lambda b, t: (0, 0)),
            pl.BlockSpec((1, 1, D), lambda b, t: (t, 0, 0)),
        ],
        out_specs=[
            pl.BlockSpec((1, 1, N, D), lambda b, t: (b, t, 0, 0)),
            pl.BlockSpec((1, N, D), lambda b, t: (b, 0, 0)),
        ],
        out_shape=[
            jax.ShapeDtypeStruct((B, T, N, D), jnp.float32),
            jax.ShapeDtypeStruct((B, N, D), jnp.float32),
        ],
    )(x, emb_W, emb_b2, pos_emb.reshape(T, 1, D))

    o_expert, h_expert = pl.pallas_call(
        functools.partial(_expert_body, n_sup=n_sup, n_exp=n_exp),
        grid=(B,),
        in_specs=[
            pl.BlockSpec((1, N, D), lambda b: (b, 0, 0)),
            pl.BlockSpec((n_sup, N, N), lambda b: (0, 0, 0)),
            pl.BlockSpec((D, n_exp), lambda b: (0, 0)),
            pl.BlockSpec((n_exp, 2 * CHEB_K * D, D), lambda b: (0, 0, 0)),
            pl.BlockSpec((n_exp, D), lambda b: (0, 0)),
        ],
        out_specs=[
            pl.BlockSpec((1, N, D), lambda b: (b, 0, 0)),
            pl.BlockSpec((1, N, D), lambda b: (b, 0, 0)),
        ],
        out_shape=[
            jax.ShapeDtypeStruct((B, N, D), jnp.float32),
            jax.ShapeDtypeStruct((B, N, D), jnp.float32),
        ],
        scratch_shapes=[pltpu.VMEM((n_sup, N, N), jnp.float32)],
    )(xt, supports, gate_W, exp_W, exp_b)

    return (o_expert, h_expert, current_inputs)


# speedup vs baseline: 1.5139x; 1.5139x over previous
import functools

import jax
import jax.numpy as jnp
from jax.experimental import pallas as pl
from jax.experimental.pallas import tpu as pltpu

CHEB_K = 3
TOP_K = 2


def _r32(v):
    return v.astype(jnp.bfloat16).astype(jnp.float32)


def _bdot(a, b):
    return jnp.dot(a.astype(jnp.bfloat16), b.astype(jnp.bfloat16),
                   preferred_element_type=jnp.float32)


def _embed_body(x_ref, emb_w_ref, emb_b_ref, pos_ref, out_ref, xt_ref, *, T, TB):
    tb = pl.program_id(1)
    n_tb = T // TB
    wr = _r32(emb_w_ref[...])
    C = wr.shape[0]
    acc = None
    for i in range(TB):
        xb = _r32(x_ref[0, i])
        val = xb[:, 0:1] * wr[0:1, :]
        for c in range(1, C):
            val = val + xb[:, c : c + 1] * wr[c : c + 1, :]
        val = val + emb_b_ref[0:1, :] + pos_ref[i]
        out_ref[0, i] = val
        acc = val if acc is None else acc + val

    @pl.when(tb == 0)
    def _init():
        xt_ref[0] = acc

    @pl.when((tb > 0) & (tb < n_tb - 1))
    def _acc():
        xt_ref[0] += acc

    @pl.when(tb == n_tb - 1)
    def _fin():
        xt_ref[0] = (xt_ref[0] + acc) / float(T)


def _expert_body(xt_ref, sup_ref, gate_w_ref, exp_w_ref, exp_b_ref,
                 o_ref, h_ref, t2_ref, *, n_sup, n_exp):
    f32 = jnp.float32
    b = pl.program_id(0)
    xt = xt_ref[0]
    N = xt.shape[0]

    @pl.when(b == 0)
    def _build_t2():
        row = jax.lax.broadcasted_iota(jnp.int32, (N, N), 0)
        col = jax.lax.broadcasted_iota(jnp.int32, (N, N), 1)
        eye = (row == col).astype(f32)
        for s in range(n_sup):
            A = sup_ref[s]
            t2_ref[s] = 2.0 * _bdot(A, A) - eye

    chunks = []
    for s in range(n_sup):
        A = sup_ref[s]
        z1 = _bdot(A, xt)
        z2 = _bdot(t2_ref[s], xt)
        chunks.extend([_r32(xt), z1, z2])
    xg = jnp.concatenate(chunks, axis=1)

    gate = _bdot(xt, gate_w_ref[...])
    iota = jax.lax.broadcasted_iota(jnp.int32, (N, n_exp), 1)
    m1 = jnp.max(gate, axis=1, keepdims=True)
    idx1 = jnp.min(jnp.where(gate == m1, iota, n_exp), axis=1, keepdims=True)
    masked = jnp.where(iota == idx1, -jnp.inf, gate)
    m2 = jnp.max(masked, axis=1, keepdims=True)
    idx2 = jnp.min(jnp.where(masked == m2, iota, n_exp), axis=1, keepdims=True)
    e1 = jnp.exp(m2 - m1)
    denom = 1.0 + e1
    w1 = 1.0 / denom
    w2 = e1 / denom

    o = jnp.zeros_like(xt)
    for e in range(n_exp):
        oe = _bdot(xg, exp_w_ref[e])
        oe = oe + exp_b_ref[e : e + 1, :]
        coef = jnp.where(idx1 == e, w1, 0.0) + jnp.where(idx2 == e, w2, 0.0)
        o = o + coef * oe
    o_ref[0] = o
    h_ref[0] = jnp.tanh(o)


@jax.jit
def kernel(x, y_cov, supports, emb_W, emb_b, pos_emb, gate_W, exp_W, exp_b):
    B, T, N, C = x.shape
    D = emb_W.shape[1]
    n_sup = supports.shape[0]
    n_exp = exp_W.shape[0]
    emb_b2 = emb_b.reshape(1, D)

    TB = 6
    current_inputs, xt = pl.pallas_call(
        functools.partial(_embed_body, T=T, TB=TB),
        grid=(B, T // TB),
        in_specs=[
            pl.BlockSpec((1, TB, N, C), lambda b, t: (b, t, 0, 0)),
            pl.BlockSpec((C, D), lambda b, t: (0, 0)),
            pl.BlockSpec((1, D), lambda b, t: (0, 0)),
            pl.BlockSpec((TB, 1, D), lambda b, t: (t, 0, 0)),
        ],
        out_specs=[
            pl.BlockSpec((1, TB, N, D), lambda b, t: (b, t, 0, 0)),
            pl.BlockSpec((1, N, D), lambda b, t: (b, 0, 0)),
        ],
        out_shape=[
            jax.ShapeDtypeStruct((B, T, N, D), jnp.float32),
            jax.ShapeDtypeStruct((B, N, D), jnp.float32),
        ],
    )(x, emb_W, emb_b2, pos_emb.reshape(T, 1, D))

    o_expert, h_expert = pl.pallas_call(
        functools.partial(_expert_body, n_sup=n_sup, n_exp=n_exp),
        grid=(B,),
        in_specs=[
            pl.BlockSpec((1, N, D), lambda b: (b, 0, 0)),
            pl.BlockSpec((n_sup, N, N), lambda b: (0, 0, 0)),
            pl.BlockSpec((D, n_exp), lambda b: (0, 0)),
            pl.BlockSpec((n_exp, 2 * CHEB_K * D, D), lambda b: (0, 0, 0)),
            pl.BlockSpec((n_exp, D), lambda b: (0, 0)),
        ],
        out_specs=[
            pl.BlockSpec((1, N, D), lambda b: (b, 0, 0)),
            pl.BlockSpec((1, N, D), lambda b: (b, 0, 0)),
        ],
        out_shape=[
            jax.ShapeDtypeStruct((B, N, D), jnp.float32),
            jax.ShapeDtypeStruct((B, N, D), jnp.float32),
        ],
        scratch_shapes=[pltpu.VMEM((n_sup, N, N), jnp.float32)],
    )(xt, supports, gate_W, exp_W, exp_b)

    return (o_expert, h_expert, current_inputs)


# speedup vs baseline: 1.5644x; 1.0333x over previous
import functools

import jax
import jax.numpy as jnp
from jax.experimental import pallas as pl
from jax.experimental.pallas import tpu as pltpu

CHEB_K = 3
TOP_K = 2


def _r32(v):
    return v.astype(jnp.bfloat16).astype(jnp.float32)


def _bdot(a, b):
    return jnp.dot(a.astype(jnp.bfloat16), b.astype(jnp.bfloat16),
                   preferred_element_type=jnp.float32)


def _embed_body(x_ref, emb_w_ref, emb_b_ref, pos_ref, out_ref, xt_ref, *, T, TB):
    tb = pl.program_id(1)
    n_tb = T // TB
    wr = _r32(emb_w_ref[...])
    C = wr.shape[0]
    acc = None
    for i in range(TB):
        xb = _r32(x_ref[0, i])
        val = xb[:, 0:1] * wr[0:1, :]
        for c in range(1, C):
            val = val + xb[:, c : c + 1] * wr[c : c + 1, :]
        val = val + emb_b_ref[0:1, :] + pos_ref[i]
        out_ref[0, i] = val
        acc = val if acc is None else acc + val

    @pl.when(tb == 0)
    def _init():
        xt_ref[0] = acc

    @pl.when((tb > 0) & (tb < n_tb - 1))
    def _acc():
        xt_ref[0] += acc

    @pl.when(tb == n_tb - 1)
    def _fin():
        xt_ref[0] = (xt_ref[0] + acc) / float(T)


def _expert_body(xt_ref, sup_ref, gate_w_ref, exp_w_ref, exp_b_ref,
                 o_ref, h_ref, t2_ref, *, n_sup, n_exp):
    f32 = jnp.float32
    b = pl.program_id(0)
    xt = xt_ref[0]
    N = xt.shape[0]

    @pl.when(b == 0)
    def _build_t2():
        row = jax.lax.broadcasted_iota(jnp.int32, (N, N), 0)
        col = jax.lax.broadcasted_iota(jnp.int32, (N, N), 1)
        eye = (row == col).astype(f32)
        for s in range(n_sup):
            A = sup_ref[s]
            t2_ref[s] = 2.0 * _bdot(A, A) - eye

    chunks = []
    for s in range(n_sup):
        A = sup_ref[s]
        z1 = _bdot(A, xt)
        z2 = _bdot(t2_ref[s], xt)
        chunks.extend([_r32(xt), z1, z2])
    xg = jnp.concatenate(chunks, axis=1)

    gate = _bdot(xt, gate_w_ref[...])
    iota = jax.lax.broadcasted_iota(jnp.int32, (N, n_exp), 1)
    m1 = jnp.max(gate, axis=1, keepdims=True)
    idx1 = jnp.min(jnp.where(gate == m1, iota, n_exp), axis=1, keepdims=True)
    masked = jnp.where(iota == idx1, -jnp.inf, gate)
    m2 = jnp.max(masked, axis=1, keepdims=True)
    idx2 = jnp.min(jnp.where(masked == m2, iota, n_exp), axis=1, keepdims=True)
    e1 = jnp.exp(m2 - m1)
    denom = 1.0 + e1
    w1 = 1.0 / denom
    w2 = e1 / denom

    o = jnp.zeros_like(xt)
    for e in range(n_exp):
        oe = _bdot(xg, exp_w_ref[e])
        oe = oe + exp_b_ref[e : e + 1, :]
        coef = jnp.where(idx1 == e, w1, 0.0) + jnp.where(idx2 == e, w2, 0.0)
        o = o + coef * oe
    o_ref[0] = o
    h_ref[0] = jnp.tanh(o)


@jax.jit
def kernel(x, y_cov, supports, emb_W, emb_b, pos_emb, gate_W, exp_W, exp_b):
    B, T, N, C = x.shape
    D = emb_W.shape[1]
    n_sup = supports.shape[0]
    n_exp = exp_W.shape[0]
    emb_b2 = emb_b.reshape(1, D)

    TB = 12
    current_inputs, xt = pl.pallas_call(
        functools.partial(_embed_body, T=T, TB=TB),
        grid=(B, T // TB),
        in_specs=[
            pl.BlockSpec((1, TB, N, C), lambda b, t: (b, t, 0, 0)),
            pl.BlockSpec((C, D), lambda b, t: (0, 0)),
            pl.BlockSpec((1, D), lambda b, t: (0, 0)),
            pl.BlockSpec((TB, 1, D), lambda b, t: (t, 0, 0)),
        ],
        out_specs=[
            pl.BlockSpec((1, TB, N, D), lambda b, t: (b, t, 0, 0)),
            pl.BlockSpec((1, N, D), lambda b, t: (b, 0, 0)),
        ],
        out_shape=[
            jax.ShapeDtypeStruct((B, T, N, D), jnp.float32),
            jax.ShapeDtypeStruct((B, N, D), jnp.float32),
        ],
    )(x, emb_W, emb_b2, pos_emb.reshape(T, 1, D))

    o_expert, h_expert = pl.pallas_call(
        functools.partial(_expert_body, n_sup=n_sup, n_exp=n_exp),
        grid=(B,),
        in_specs=[
            pl.BlockSpec((1, N, D), lambda b: (b, 0, 0)),
            pl.BlockSpec((n_sup, N, N), lambda b: (0, 0, 0)),
            pl.BlockSpec((D, n_exp), lambda b: (0, 0)),
            pl.BlockSpec((n_exp, 2 * CHEB_K * D, D), lambda b: (0, 0, 0)),
            pl.BlockSpec((n_exp, D), lambda b: (0, 0)),
        ],
        out_specs=[
            pl.BlockSpec((1, N, D), lambda b: (b, 0, 0)),
            pl.BlockSpec((1, N, D), lambda b: (b, 0, 0)),
        ],
        out_shape=[
            jax.ShapeDtypeStruct((B, N, D), jnp.float32),
            jax.ShapeDtypeStruct((B, N, D), jnp.float32),
        ],
        scratch_shapes=[pltpu.VMEM((n_sup, N, N), jnp.float32)],
    )(xt, supports, gate_W, exp_W, exp_b)

    return (o_expert, h_expert, current_inputs)


# speedup vs baseline: 2.6084x; 1.6674x over previous
import functools

import jax
import jax.numpy as jnp
from jax.experimental import pallas as pl
from jax.experimental.pallas import tpu as pltpu

CHEB_K = 3
TOP_K = 2


def _r32(v):
    return v.astype(jnp.bfloat16).astype(jnp.float32)


def _bdot(a, b):
    return jnp.dot(a.astype(jnp.bfloat16), b.astype(jnp.bfloat16),
                   preferred_element_type=jnp.float32)


def _embed_body(x_ref, emb_w_ref, emb_b_ref, pos_ref, out_ref, xt_ref, *, T, TB):
    tb = pl.program_id(1)
    n_tb = T // TB
    wr = _r32(emb_w_ref[...])
    C = wr.shape[0]
    acc = None
    for i in range(TB):
        xb = _r32(jnp.transpose(x_ref[0, i]))
        val = xb[:, 0:1] * wr[0:1, :]
        for c in range(1, C):
            val = val + xb[:, c : c + 1] * wr[c : c + 1, :]
        val = val + emb_b_ref[0:1, :] + pos_ref[i]
        out_ref[0, i] = val
        acc = val if acc is None else acc + val

    @pl.when(tb == 0)
    def _init():
        xt_ref[0] = acc

    @pl.when((tb > 0) & (tb < n_tb - 1))
    def _acc():
        xt_ref[0] += acc

    @pl.when(tb == n_tb - 1)
    def _fin():
        xt_ref[0] = (xt_ref[0] + acc) / float(T)


def _expert_body(xt_ref, sup_ref, gate_w_ref, exp_w_ref, exp_b_ref,
                 o_ref, h_ref, t2_ref, *, n_sup, n_exp):
    f32 = jnp.float32
    b = pl.program_id(0)
    xt = xt_ref[0]
    N = xt.shape[0]

    @pl.when(b == 0)
    def _build_t2():
        row = jax.lax.broadcasted_iota(jnp.int32, (N, N), 0)
        col = jax.lax.broadcasted_iota(jnp.int32, (N, N), 1)
        eye = (row == col).astype(f32)
        for s in range(n_sup):
            A = sup_ref[s]
            t2_ref[s] = 2.0 * _bdot(A, A) - eye

    chunks = []
    for s in range(n_sup):
        A = sup_ref[s]
        z1 = _bdot(A, xt)
        z2 = _bdot(t2_ref[s], xt)
        chunks.extend([_r32(xt), z1, z2])
    xg = jnp.concatenate(chunks, axis=1)

    gate = _bdot(xt, gate_w_ref[...])
    iota = jax.lax.broadcasted_iota(jnp.int32, (N, n_exp), 1)
    m1 = jnp.max(gate, axis=1, keepdims=True)
    idx1 = jnp.min(jnp.where(gate == m1, iota, n_exp), axis=1, keepdims=True)
    masked = jnp.where(iota == idx1, -jnp.inf, gate)
    m2 = jnp.max(masked, axis=1, keepdims=True)
    idx2 = jnp.min(jnp.where(masked == m2, iota, n_exp), axis=1, keepdims=True)
    e1 = jnp.exp(m2 - m1)
    denom = 1.0 + e1
    w1 = 1.0 / denom
    w2 = e1 / denom

    o = jnp.zeros_like(xt)
    for e in range(n_exp):
        oe = _bdot(xg, exp_w_ref[e])
        oe = oe + exp_b_ref[e : e + 1, :]
        coef = jnp.where(idx1 == e, w1, 0.0) + jnp.where(idx2 == e, w2, 0.0)
        o = o + coef * oe
    o_ref[0] = o
    h_ref[0] = jnp.tanh(o)


@jax.jit
def kernel(x, y_cov, supports, emb_W, emb_b, pos_emb, gate_W, exp_W, exp_b):
    B, T, N, C = x.shape
    D = emb_W.shape[1]
    n_sup = supports.shape[0]
    n_exp = exp_W.shape[0]
    emb_b2 = emb_b.reshape(1, D)

    TB = 12
    current_inputs, xt = pl.pallas_call(
        functools.partial(_embed_body, T=T, TB=TB),
        grid=(B, T // TB),
        in_specs=[
            pl.BlockSpec((1, TB, C, N), lambda b, t: (b, t, 0, 0)),
            pl.BlockSpec((C, D), lambda b, t: (0, 0)),
            pl.BlockSpec((1, D), lambda b, t: (0, 0)),
            pl.BlockSpec((TB, 1, D), lambda b, t: (t, 0, 0)),
        ],
        out_specs=[
            pl.BlockSpec((1, TB, N, D), lambda b, t: (b, t, 0, 0)),
            pl.BlockSpec((1, N, D), lambda b, t: (b, 0, 0)),
        ],
        out_shape=[
            jax.ShapeDtypeStruct((B, T, N, D), jnp.float32),
            jax.ShapeDtypeStruct((B, N, D), jnp.float32),
        ],
    )(jnp.swapaxes(x, 2, 3), emb_W, emb_b2, pos_emb.reshape(T, 1, D))

    o_expert, h_expert = pl.pallas_call(
        functools.partial(_expert_body, n_sup=n_sup, n_exp=n_exp),
        grid=(B,),
        in_specs=[
            pl.BlockSpec((1, N, D), lambda b: (b, 0, 0)),
            pl.BlockSpec((n_sup, N, N), lambda b: (0, 0, 0)),
            pl.BlockSpec((D, n_exp), lambda b: (0, 0)),
            pl.BlockSpec((n_exp, 2 * CHEB_K * D, D), lambda b: (0, 0, 0)),
            pl.BlockSpec((n_exp, D), lambda b: (0, 0)),
        ],
        out_specs=[
            pl.BlockSpec((1, N, D), lambda b: (b, 0, 0)),
            pl.BlockSpec((1, N, D), lambda b: (b, 0, 0)),
        ],
        out_shape=[
            jax.ShapeDtypeStruct((B, N, D), jnp.float32),
            jax.ShapeDtypeStruct((B, N, D), jnp.float32),
        ],
        scratch_shapes=[pltpu.VMEM((n_sup, N, N), jnp.float32)],
    )(xt, supports, gate_W, exp_W, exp_b)

    return (o_expert, h_expert, current_inputs)


# speedup vs baseline: 2.6334x; 1.0096x over previous
import functools

import jax
import jax.numpy as jnp
from jax.experimental import pallas as pl
from jax.experimental.pallas import tpu as pltpu

CHEB_K = 3
TOP_K = 2


def _r32(v):
    return v.astype(jnp.bfloat16).astype(jnp.float32)


def _bdot(a, b):
    return jnp.dot(a.astype(jnp.bfloat16), b.astype(jnp.bfloat16),
                   preferred_element_type=jnp.float32)


def _fused_body(x_ref, emb_w_ref, emb_b_ref, pos_ref, sup_ref, gate_w_ref,
                exp_w_ref, exp_b_ref, o_ref, h_ref, out_ref,
                xt_ref, t2_ref, *, T, TB, n_sup, n_exp):
    f32 = jnp.float32
    b = pl.program_id(0)
    tb = pl.program_id(1)
    n_tb = T // TB

    @pl.when(tb < n_tb)
    def _embed():
        wr = _r32(emb_w_ref[...])
        C = wr.shape[0]
        acc = None
        for i in range(TB):
            xb = _r32(jnp.transpose(x_ref[0, i]))
            val = xb[:, 0:1] * wr[0:1, :]
            for c in range(1, C):
                val = val + xb[:, c : c + 1] * wr[c : c + 1, :]
            val = val + emb_b_ref[0:1, :] + pos_ref[i]
            out_ref[0, i] = val
            acc = val if acc is None else acc + val

        @pl.when(tb == 0)
        def _init():
            xt_ref[...] = acc

        @pl.when(tb > 0)
        def _acc():
            xt_ref[...] += acc

    @pl.when((b == 0) & (tb == 0))
    def _build_t2():
        N = sup_ref.shape[1]
        row = jax.lax.broadcasted_iota(jnp.int32, (N, N), 0)
        col = jax.lax.broadcasted_iota(jnp.int32, (N, N), 1)
        eye = (row == col).astype(f32)
        for s in range(n_sup):
            A = sup_ref[s]
            t2_ref[s] = 2.0 * _bdot(A, A) - eye

    @pl.when(tb == n_tb)
    def _expert():
        xt = xt_ref[...] / float(T)
        N = xt.shape[0]

        chunks = []
        for s in range(n_sup):
            A = sup_ref[s]
            z1 = _bdot(A, xt)
            z2 = _bdot(t2_ref[s], xt)
            chunks.extend([_r32(xt), z1, z2])
        xg = jnp.concatenate(chunks, axis=1)

        gate = _bdot(xt, gate_w_ref[...])
        iota = jax.lax.broadcasted_iota(jnp.int32, (N, n_exp), 1)
        m1 = jnp.max(gate, axis=1, keepdims=True)
        idx1 = jnp.min(jnp.where(gate == m1, iota, n_exp), axis=1, keepdims=True)
        masked = jnp.where(iota == idx1, -jnp.inf, gate)
        m2 = jnp.max(masked, axis=1, keepdims=True)
        idx2 = jnp.min(jnp.where(masked == m2, iota, n_exp), axis=1, keepdims=True)
        e1 = jnp.exp(m2 - m1)
        denom = 1.0 + e1
        w1 = 1.0 / denom
        w2 = e1 / denom

        o = jnp.zeros_like(xt)
        for e in range(n_exp):
            oe = _bdot(xg, exp_w_ref[e])
            oe = oe + exp_b_ref[e : e + 1, :]
            coef = jnp.where(idx1 == e, w1, 0.0) + jnp.where(idx2 == e, w2, 0.0)
            o = o + coef * oe
        o_ref[0] = o
        h_ref[0] = jnp.tanh(o)


@jax.jit
def kernel(x, y_cov, supports, emb_W, emb_b, pos_emb, gate_W, exp_W, exp_b):
    B, T, N, C = x.shape
    D = emb_W.shape[1]
    n_sup = supports.shape[0]
    n_exp = exp_W.shape[0]
    TB = 12
    n_tb = T // TB
    last = n_tb - 1

    o_expert, h_expert, current_inputs = pl.pallas_call(
        functools.partial(_fused_body, T=T, TB=TB, n_sup=n_sup, n_exp=n_exp),
        grid=(B, n_tb + 1),
        in_specs=[
            pl.BlockSpec((1, TB, C, N), lambda b, t: (b, jnp.minimum(t, last), 0, 0)),
            pl.BlockSpec((C, D), lambda b, t: (0, 0)),
            pl.BlockSpec((1, D), lambda b, t: (0, 0)),
            pl.BlockSpec((TB, 1, D), lambda b, t: (jnp.minimum(t, last), 0, 0)),
            pl.BlockSpec((n_sup, N, N), lambda b, t: (0, 0, 0)),
            pl.BlockSpec((D, n_exp), lambda b, t: (0, 0)),
            pl.BlockSpec((n_exp, 2 * CHEB_K * D, D), lambda b, t: (0, 0, 0)),
            pl.BlockSpec((n_exp, D), lambda b, t: (0, 0)),
        ],
        out_specs=[
            pl.BlockSpec((1, N, D), lambda b, t: (b, 0, 0)),
            pl.BlockSpec((1, N, D), lambda b, t: (b, 0, 0)),
            pl.BlockSpec((1, TB, N, D), lambda b, t: (b, jnp.minimum(t, last), 0, 0)),
        ],
        out_shape=[
            jax.ShapeDtypeStruct((B, N, D), jnp.float32),
            jax.ShapeDtypeStruct((B, N, D), jnp.float32),
            jax.ShapeDtypeStruct((B, T, N, D), jnp.float32),
        ],
        scratch_shapes=[
            pltpu.VMEM((N, D), jnp.float32),
            pltpu.VMEM((n_sup, N, N), jnp.float32),
        ],
    )(jnp.swapaxes(x, 2, 3), emb_W, emb_b.reshape(1, D),
      pos_emb.reshape(T, 1, D), supports, gate_W, exp_W, exp_b)

    return (o_expert, h_expert, current_inputs)


# speedup vs baseline: 2.7059x; 1.0275x over previous
import functools

import jax
import jax.numpy as jnp
from jax.experimental import pallas as pl
from jax.experimental.pallas import tpu as pltpu

CHEB_K = 3
TOP_K = 2


def _r32(v):
    return v.astype(jnp.bfloat16).astype(jnp.float32)


def _bdot(a, b):
    return jnp.dot(a.astype(jnp.bfloat16), b.astype(jnp.bfloat16),
                   preferred_element_type=jnp.float32)


def _fused_body(x_ref, emb_w_ref, emb_b_ref, pos_ref, sup_ref, gate_w_ref,
                exp_w_ref, exp_b_ref, o_ref, h_ref, out_ref,
                xt_ref, a_ref, t2_ref, *, T, TB, n_sup, n_exp):
    f32 = jnp.float32
    b = pl.program_id(0)
    tb = pl.program_id(1)
    n_tb = T // TB

    @pl.when(tb < n_tb)
    def _embed():
        wr = _r32(emb_w_ref[...])
        C = wr.shape[0]
        acc = None
        for i in range(TB):
            xb = _r32(jnp.transpose(x_ref[0, i]))
            bias = emb_b_ref[0:1, :] + pos_ref[i]
            val = xb[:, 0:1] * wr[0:1, :] + bias
            for c in range(1, C):
                val = val + xb[:, c : c + 1] * wr[c : c + 1, :]
            out_ref[0, i] = val
            acc = val if acc is None else acc + val

        @pl.when(tb == 0)
        def _init():
            xt_ref[...] = acc

        @pl.when(tb > 0)
        def _acc():
            xt_ref[...] += acc

    @pl.when((b == 0) & (tb == 0))
    def _build_t2():
        N = sup_ref.shape[1]
        row = jax.lax.broadcasted_iota(jnp.int32, (N, N), 0)
        col = jax.lax.broadcasted_iota(jnp.int32, (N, N), 1)
        eye = (row == col).astype(f32)
        for s in range(n_sup):
            a_bf = sup_ref[s].astype(jnp.bfloat16)
            a_ref[s] = a_bf
            t2 = 2.0 * jnp.dot(a_bf, a_bf, preferred_element_type=f32) - eye
            t2_ref[s] = t2.astype(jnp.bfloat16)

    @pl.when(tb == n_tb)
    def _expert():
        bf = jnp.bfloat16
        xt = xt_ref[...] / float(T)
        N = xt.shape[0]
        xt_bf = xt.astype(bf)

        chunks = []
        for s in range(n_sup):
            z1 = jnp.dot(a_ref[s], xt_bf, preferred_element_type=f32)
            z2 = jnp.dot(t2_ref[s], xt_bf, preferred_element_type=f32)
            chunks.extend([xt_bf, z1.astype(bf), z2.astype(bf)])
        xg = jnp.concatenate(chunks, axis=1)

        gate = jnp.dot(xt_bf, gate_w_ref[...].astype(bf),
                       preferred_element_type=f32)
        iota = jax.lax.broadcasted_iota(jnp.int32, (N, n_exp), 1)
        m1 = jnp.max(gate, axis=1, keepdims=True)
        idx1 = jnp.min(jnp.where(gate == m1, iota, n_exp), axis=1, keepdims=True)
        masked = jnp.where(iota == idx1, -jnp.inf, gate)
        m2 = jnp.max(masked, axis=1, keepdims=True)
        idx2 = jnp.min(jnp.where(masked == m2, iota, n_exp), axis=1, keepdims=True)
        e1 = jnp.exp(m2 - m1)
        denom = 1.0 + e1
        w1 = 1.0 / denom
        w2 = e1 / denom

        o = jnp.zeros_like(xt)
        for e in range(n_exp):
            oe = jnp.dot(xg, exp_w_ref[e].astype(bf), preferred_element_type=f32)
            oe = oe + exp_b_ref[e : e + 1, :]
            coef = jnp.where(idx1 == e, w1, 0.0) + jnp.where(idx2 == e, w2, 0.0)
            o = o + coef * oe
        o_ref[0] = o
        h_ref[0] = jnp.tanh(o)


@jax.jit
def kernel(x, y_cov, supports, emb_W, emb_b, pos_emb, gate_W, exp_W, exp_b):
    B, T, N, C = x.shape
    D = emb_W.shape[1]
    n_sup = supports.shape[0]
    n_exp = exp_W.shape[0]
    TB = 12
    n_tb = T // TB
    last = n_tb - 1

    o_expert, h_expert, current_inputs = pl.pallas_call(
        functools.partial(_fused_body, T=T, TB=TB, n_sup=n_sup, n_exp=n_exp),
        grid=(B, n_tb + 1),
        in_specs=[
            pl.BlockSpec((1, TB, C, N), lambda b, t: (b, jnp.minimum(t, last), 0, 0)),
            pl.BlockSpec((C, D), lambda b, t: (0, 0)),
            pl.BlockSpec((1, D), lambda b, t: (0, 0)),
            pl.BlockSpec((TB, 1, D), lambda b, t: (jnp.minimum(t, last), 0, 0)),
            pl.BlockSpec((n_sup, N, N), lambda b, t: (0, 0, 0)),
            pl.BlockSpec((D, n_exp), lambda b, t: (0, 0)),
            pl.BlockSpec((n_exp, 2 * CHEB_K * D, D), lambda b, t: (0, 0, 0)),
            pl.BlockSpec((n_exp, D), lambda b, t: (0, 0)),
        ],
        out_specs=[
            pl.BlockSpec((1, N, D), lambda b, t: (b, 0, 0)),
            pl.BlockSpec((1, N, D), lambda b, t: (b, 0, 0)),
            pl.BlockSpec((1, TB, N, D), lambda b, t: (b, jnp.minimum(t, last), 0, 0)),
        ],
        out_shape=[
            jax.ShapeDtypeStruct((B, N, D), jnp.float32),
            jax.ShapeDtypeStruct((B, N, D), jnp.float32),
            jax.ShapeDtypeStruct((B, T, N, D), jnp.float32),
        ],
        scratch_shapes=[
            pltpu.VMEM((N, D), jnp.float32),
            pltpu.VMEM((n_sup, N, N), jnp.bfloat16),
            pltpu.VMEM((n_sup, N, N), jnp.bfloat16),
        ],
    )(jnp.swapaxes(x, 2, 3), emb_W, emb_b.reshape(1, D),
      pos_emb.reshape(T, 1, D), supports, gate_W, exp_W, exp_b)

    return (o_expert, h_expert, current_inputs)


# speedup vs baseline: 2.9900x; 1.1050x over previous
import functools

import jax
import jax.numpy as jnp
from jax.experimental import pallas as pl
from jax.experimental.pallas import tpu as pltpu

CHEB_K = 3
TOP_K = 2


def _fused_body(x_ref, emb_w_ref, emb_b_ref, pos_ref, sup_ref, gate_w_ref,
                exp_w_ref, exp_b_ref, o_ref, h_ref, out_ref,
                xt_ref, a_ref, t2_ref, *, T, TB, n_sup, n_exp):
    f32 = jnp.float32
    b = pl.program_id(0)
    tb = pl.program_id(1)
    n_tb = T // TB

    @pl.when(tb < n_tb)
    def _embed():
        wr = emb_w_ref[...].astype(jnp.bfloat16)
        acc = None
        for i in range(TB):
            xb = jnp.transpose(x_ref[0, i].astype(jnp.bfloat16))
            bias = emb_b_ref[0:1, :] + pos_ref[i]
            val = jnp.dot(xb, wr, preferred_element_type=f32) + bias
            out_ref[0, i] = val
            acc = val if acc is None else acc + val

        @pl.when(tb == 0)
        def _init():
            xt_ref[...] = acc

        @pl.when(tb > 0)
        def _acc():
            xt_ref[...] += acc

    @pl.when((b == 0) & (tb == 0))
    def _build_t2():
        N = sup_ref.shape[1]
        row = jax.lax.broadcasted_iota(jnp.int32, (N, N), 0)
        col = jax.lax.broadcasted_iota(jnp.int32, (N, N), 1)
        eye = (row == col).astype(f32)
        for s in range(n_sup):
            a_bf = sup_ref[s].astype(jnp.bfloat16)
            a_ref[s] = a_bf
            t2 = 2.0 * jnp.dot(a_bf, a_bf, preferred_element_type=f32) - eye
            t2_ref[s] = t2.astype(jnp.bfloat16)

    @pl.when(tb == n_tb)
    def _expert():
        bf = jnp.bfloat16
        xt = xt_ref[...] / float(T)
        N = xt.shape[0]
        xt_bf = xt.astype(bf)

        chunks = []
        for s in range(n_sup):
            z1 = jnp.dot(a_ref[s], xt_bf, preferred_element_type=f32)
            z2 = jnp.dot(t2_ref[s], xt_bf, preferred_element_type=f32)
            chunks.extend([xt_bf, z1.astype(bf), z2.astype(bf)])
        xg = jnp.concatenate(chunks, axis=1)

        gate = jnp.dot(xt_bf, gate_w_ref[...].astype(bf),
                       preferred_element_type=f32)
        iota = jax.lax.broadcasted_iota(jnp.int32, (N, n_exp), 1)
        m1 = jnp.max(gate, axis=1, keepdims=True)
        idx1 = jnp.min(jnp.where(gate == m1, iota, n_exp), axis=1, keepdims=True)
        masked = jnp.where(iota == idx1, -jnp.inf, gate)
        m2 = jnp.max(masked, axis=1, keepdims=True)
        idx2 = jnp.min(jnp.where(masked == m2, iota, n_exp), axis=1, keepdims=True)
        e1 = jnp.exp(m2 - m1)
        denom = 1.0 + e1
        w1 = 1.0 / denom
        w2 = e1 / denom

        o = jnp.zeros_like(xt)
        for e in range(n_exp):
            oe = jnp.dot(xg, exp_w_ref[e].astype(bf), preferred_element_type=f32)
            oe = oe + exp_b_ref[e : e + 1, :]
            coef = jnp.where(idx1 == e, w1, 0.0) + jnp.where(idx2 == e, w2, 0.0)
            o = o + coef * oe
        o_ref[0] = o
        h_ref[0] = jnp.tanh(o)


@jax.jit
def kernel(x, y_cov, supports, emb_W, emb_b, pos_emb, gate_W, exp_W, exp_b):
    B, T, N, C = x.shape
    D = emb_W.shape[1]
    n_sup = supports.shape[0]
    n_exp = exp_W.shape[0]
    TB = 12
    n_tb = T // TB
    last = n_tb - 1

    o_expert, h_expert, current_inputs = pl.pallas_call(
        functools.partial(_fused_body, T=T, TB=TB, n_sup=n_sup, n_exp=n_exp),
        grid=(B, n_tb + 1),
        in_specs=[
            pl.BlockSpec((1, TB, C, N), lambda b, t: (b, jnp.minimum(t, last), 0, 0)),
            pl.BlockSpec((C, D), lambda b, t: (0, 0)),
            pl.BlockSpec((1, D), lambda b, t: (0, 0)),
            pl.BlockSpec((TB, 1, D), lambda b, t: (jnp.minimum(t, last), 0, 0)),
            pl.BlockSpec((n_sup, N, N), lambda b, t: (0, 0, 0)),
            pl.BlockSpec((D, n_exp), lambda b, t: (0, 0)),
            pl.BlockSpec((n_exp, 2 * CHEB_K * D, D), lambda b, t: (0, 0, 0)),
            pl.BlockSpec((n_exp, D), lambda b, t: (0, 0)),
        ],
        out_specs=[
            pl.BlockSpec((1, N, D), lambda b, t: (b, 0, 0)),
            pl.BlockSpec((1, N, D), lambda b, t: (b, 0, 0)),
            pl.BlockSpec((1, TB, N, D), lambda b, t: (b, jnp.minimum(t, last), 0, 0)),
        ],
        out_shape=[
            jax.ShapeDtypeStruct((B, N, D), jnp.float32),
            jax.ShapeDtypeStruct((B, N, D), jnp.float32),
            jax.ShapeDtypeStruct((B, T, N, D), jnp.float32),
        ],
        scratch_shapes=[
            pltpu.VMEM((N, D), jnp.float32),
            pltpu.VMEM((n_sup, N, N), jnp.bfloat16),
            pltpu.VMEM((n_sup, N, N), jnp.bfloat16),
        ],
    )(jnp.swapaxes(x, 2, 3), emb_W, emb_b.reshape(1, D),
      pos_emb.reshape(T, 1, D), supports, gate_W, exp_W, exp_b)

    return (o_expert, h_expert, current_inputs)
